# Initial kernel scaffold; baseline (speedup 1.0000x reference)
import jax
import jax.numpy as jnp
from jax.experimental import pallas as pl


def kernel(position_weights):
    raise NotImplementedError("write your pallas kernel here")


# speedup vs baseline: 1.1889x; 1.1889x over previous
import jax
import jax.numpy as jnp
from jax.experimental import pallas as pl

_OFFSET = 2
_MAX_POS = 2048
_D_MODEL = 1024


def _slice_copy_kernel(in_ref, out_ref):
    out_ref[...] = in_ref[pl.ds(_OFFSET, _MAX_POS), :]


def kernel(position_weights):
    return pl.pallas_call(
        _slice_copy_kernel,
        out_shape=jax.ShapeDtypeStruct((_MAX_POS, _D_MODEL), jnp.float32),
    )(position_weights)
